# Initial kernel scaffold; baseline (speedup 1.0000x reference)
import jax
import jax.numpy as jnp
from jax.experimental import pallas as pl


def kernel(x):
    raise NotImplementedError("write your pallas kernel here")


# speedup vs baseline: 3.6491x; 3.6491x over previous
import jax
import jax.numpy as jnp
from jax.experimental import pallas as pl


def _build_idx_list():
    num_candidates = 16
    indices = [0, 1, 2, 3, 4, 5, 6, 7, 8]
    base_idx = 9
    for i in range(num_candidates - 1):
        indices += [6, 7, base_idx + i]
    indices += [0, 3, 6, 1, 4, 7, 2, 5, 8]
    for i in range(num_candidates - 1):
        indices += [2, 5, base_idx + i]
    return indices


_IDX = _build_idx_list()


def _merge_runs(idx):
    runs = []
    o_start, s_start, length = 0, idx[0], 1
    for j in range(1, len(idx)):
        if idx[j] == s_start + length:
            length += 1
        else:
            runs.append((o_start, s_start, length))
            o_start, s_start, length = j, idx[j], 1
    runs.append((o_start, s_start, length))
    return runs


_RUNS = _merge_runs(_IDX)


def _copy_body(x_ref, o_ref):
    for o_start, s_start, length in _RUNS:
        o_ref[0, o_start:o_start + length] = x_ref[0, s_start:s_start + length]


def kernel(x):
    b, n, s, d = x.shape
    n_out = len(_IDX)
    chunk = 128
    grid = (b, s // chunk)

    out = pl.pallas_call(
        _copy_body,
        grid=grid,
        in_specs=[pl.BlockSpec((1, n, chunk, d), lambda i, c: (i, 0, c, 0))],
        out_specs=pl.BlockSpec((1, n_out, chunk, d), lambda i, c: (i, 0, c, 0)),
        out_shape=jax.ShapeDtypeStruct((b, n_out, s, d), x.dtype),
    )(x)
    return out.reshape(b, n_out // 3, 3, s, d)
